# baseline (speedup 1.0000x reference)
import functools

import jax
import jax.numpy as jnp
from jax import lax
from jax.experimental import pallas as pl
from jax.experimental.pallas import tpu as pltpu
from jax.experimental.pallas import tpu_sc as plsc

V = 1000000
E = 32
H = 64
C = 2
B = 4096
L = 200
H2 = H // 2
BP = B // 2

L_SEG = 40
N_SEG = L // L_SEG

_NC = 2
_NS = 16
_NW = _NC * _NS

_GRP = 128
_NGRP = (L * B) // _GRP
_NGRP_S = (L_SEG * B) // _GRP
_GRP_PER_W = _NGRP_S // _NW
_KGRP = 8
_NCHUNK = _GRP_PER_W // _KGRP


def _sc_gather_seg(table, idx_grp, seg):
    mesh = plsc.VectorSubcoreMesh(core_axis_name="c", subcore_axis_name="s")
    seg_base = seg * _NGRP_S

    @functools.partial(
        pl.kernel,
        mesh=mesh,
        out_type=jax.ShapeDtypeStruct((_NGRP_S, _GRP, E), jnp.float32),
        scratch_types=[
            pltpu.VMEM((_KGRP, _GRP), jnp.int32),
            pltpu.VMEM((_KGRP, _GRP, E), jnp.float32),
            pltpu.SemaphoreType.DMA,
        ],
        compiler_params=pltpu.CompilerParams(use_tc_tiling_on_sc=False),
    )
    def gather_kernel(table_hbm, idx_hbm, out_hbm, idx_v, rows_v, sem):
        wid = lax.axis_index("s") * _NC + lax.axis_index("c")
        base = wid * _GRP_PER_W

        def chunk_body(i, carry):
            g0 = base + i * _KGRP
            pltpu.sync_copy(idx_hbm.at[pl.ds(seg_base + g0, _KGRP)], idx_v)
            copies = []
            for j in range(_KGRP):
                copies.append(
                    pltpu.async_copy(table_hbm.at[idx_v.at[j]], rows_v.at[j], sem)
                )
            for cp in copies:
                cp.wait()
            pltpu.sync_copy(rows_v, out_hbm.at[pl.ds(g0, _KGRP)])
            return carry

        lax.fori_loop(0, _NCHUNK, chunk_body, 0)

    return gather_kernel(table, idx_grp)


def _tanh_gate(t):
    return 0.5 * t + 0.5


BPC = BP // 2


def _seg_body(emb_ref, hin_ref, cin_ref, wih_ref, whh_ref, b_ref,
              hout_ref, cout_ref, h_ref, c_ref):
    l = pl.program_id(1)

    @pl.when(l == 0)
    def _():
        h_ref[...] = hin_ref[...]
        c_ref[...] = cin_ref[...]

    x4 = emb_ref[0]
    x_t = jnp.concatenate([x4[:, :2 * E], x4[:, 2 * E:]], axis=0)
    h = h_ref[...]
    c = c_ref[...]

    gates = jnp.dot(x_t, wih_ref[...], preferred_element_type=jnp.float32)
    gates = gates + jnp.dot(h, whh_ref[...], preferred_element_type=jnp.float32)
    gates = gates + b_ref[...]

    i_g = _tanh_gate(jnp.tanh(gates[:, 0 * 2 * H:1 * 2 * H]))
    f_g = _tanh_gate(jnp.tanh(gates[:, 1 * 2 * H:2 * 2 * H]))
    g_g = jnp.tanh(gates[:, 2 * 2 * H:3 * 2 * H])
    o_g = _tanh_gate(jnp.tanh(gates[:, 3 * 2 * H:4 * 2 * H]))
    c_new = f_g * c + i_g * g_g
    h_new = o_g * jnp.tanh(c_new)
    c_ref[...] = c_new
    h_ref[...] = h_new

    @pl.when(l == L_SEG - 1)
    def _():
        hout_ref[...] = h_new
        cout_ref[...] = c_new


def _lstm_seg(emb, h_in, c_in, wih2, whh2, bias2):
    return pl.pallas_call(
        _seg_body,
        grid=(2, L_SEG),
        in_specs=[
            pl.BlockSpec((1, BPC // 2, 4 * E), lambda b, l: (l, b, 0)),
            pl.BlockSpec((BPC, 2 * H), lambda b, l: (b, 0)),
            pl.BlockSpec((BPC, 2 * H), lambda b, l: (b, 0)),
            pl.BlockSpec((2 * E, 8 * H), lambda b, l: (0, 0)),
            pl.BlockSpec((2 * H, 8 * H), lambda b, l: (0, 0)),
            pl.BlockSpec((1, 8 * H), lambda b, l: (0, 0)),
        ],
        out_specs=[
            pl.BlockSpec((BPC, 2 * H), lambda b, l: (b, 0)),
            pl.BlockSpec((BPC, 2 * H), lambda b, l: (b, 0)),
        ],
        out_shape=[
            jax.ShapeDtypeStruct((BP, 2 * H), jnp.float32),
            jax.ShapeDtypeStruct((BP, 2 * H), jnp.float32),
        ],
        scratch_shapes=[
            pltpu.VMEM((BPC, 2 * H), jnp.float32),
            pltpu.VMEM((BPC, 2 * H), jnp.float32),
        ],
        compiler_params=pltpu.CompilerParams(
            dimension_semantics=("parallel", "arbitrary"),
        ),
    )(emb, h_in, c_in, wih2, whh2, bias2)


def _fc_body(h_ref, w1_ref, b1_ref, w2_ref, b2_ref, out_ref):
    z = jnp.dot(h_ref[...], w1_ref[...], preferred_element_type=jnp.float32)
    z = jnp.maximum(z + b1_ref[...], 0.0)
    out = jnp.dot(z, w2_ref[...], preferred_element_type=jnp.float32)
    out_ref[...] = out + b2_ref[...]


def _fc(h_fin, w1_2, b1_2, w2_2, b2_2):
    return pl.pallas_call(
        _fc_body,
        out_shape=jax.ShapeDtypeStruct((BP, 2 * C), jnp.float32),
    )(h_fin, w1_2, b1_2, w2_2, b2_2)


def _pack_block_diag(w, scale):
    K = w.shape[0]
    wg = (w * scale[None, :]).reshape(K, 4, H)
    z = jnp.zeros_like(wg)
    top = jnp.concatenate([wg, z], axis=2)
    bot = jnp.concatenate([z, wg], axis=2)
    return jnp.concatenate([top, bot], axis=0).reshape(2 * K, 4 * 2 * H)


def kernel(x, table, W_ih, W_hh, b_ih, b_hh, W1, b1, W2, b2):
    x = x.astype(jnp.int32)
    idx = x.T.reshape(L, 2, BP).transpose(0, 2, 1)
    idx_grp = idx.reshape(_NGRP, _GRP)

    gs = jnp.array([0.5, 0.5, 1.0, 0.5], jnp.float32)
    scale = jnp.repeat(gs, H)
    wih2 = _pack_block_diag(W_ih.T, scale)
    whh2 = _pack_block_diag(W_hh.T, scale)
    bias = ((b_ih + b_hh) * scale).reshape(4, H)
    bias2 = jnp.concatenate([bias, bias], axis=1).reshape(1, 8 * H)

    z32 = jnp.zeros((H, H2), jnp.float32)
    w1_2 = jnp.concatenate(
        [jnp.concatenate([W1.T, z32], axis=1),
         jnp.concatenate([z32, W1.T], axis=1)], axis=0)
    b1_2 = jnp.concatenate([b1, b1]).reshape(1, 2 * H2)
    zc = jnp.zeros((H2, C), jnp.float32)
    w2_2 = jnp.concatenate(
        [jnp.concatenate([W2.T, zc], axis=1),
         jnp.concatenate([zc, W2.T], axis=1)], axis=0)
    b2_2 = jnp.concatenate([b2, b2]).reshape(1, 2 * C)

    h = jnp.zeros((BP, 2 * H), jnp.float32)
    c = jnp.zeros((BP, 2 * H), jnp.float32)
    for seg in range(N_SEG):
        emb = _sc_gather_seg(table, idx_grp, seg)
        emb = emb.reshape(L_SEG, BP // 2, 4 * E)
        h, c = _lstm_seg(emb, h, c, wih2, whh2, bias2)

    out2 = _fc(h, w1_2, b1_2, w2_2, b2_2)
    out_nat = (out2.reshape(2, 2, BPC // 2, 2 * C)
               .transpose(0, 2, 1, 3).reshape(BP, 2 * C))
    out = jnp.concatenate([out_nat[:, 0:C], out_nat[:, C:2 * C]], axis=0)
    return out[None]


# speedup vs baseline: 1.1235x; 1.1235x over previous
import functools

import jax
import jax.numpy as jnp
from jax import lax
from jax.experimental import pallas as pl
from jax.experimental.pallas import tpu as pltpu
from jax.experimental.pallas import tpu_sc as plsc

V = 1000000
E = 32
H = 64
C = 2
B = 4096
L = 200
H2 = H // 2
BP = B // 2

L_SEG = 40
N_SEG = L // L_SEG

_NC = 2
_NS = 16
_NW = _NC * _NS

_GRP = 128
_NGRP = (L * B) // _GRP
_NGRP_S = (L_SEG * B) // _GRP
_GRP_PER_W = _NGRP_S // _NW
_KGRP = 8
_NCHUNK = _GRP_PER_W // _KGRP


def _sc_gather_seg(table, idx_grp, seg):
    mesh = plsc.VectorSubcoreMesh(core_axis_name="c", subcore_axis_name="s")
    seg_base = seg * _NGRP_S

    @functools.partial(
        pl.kernel,
        mesh=mesh,
        out_type=jax.ShapeDtypeStruct((_NGRP_S, _GRP, E), jnp.float32),
        scratch_types=[
            pltpu.VMEM((_KGRP, _GRP), jnp.int32),
            pltpu.VMEM((_KGRP, _GRP, E), jnp.float32),
            pltpu.SemaphoreType.DMA,
        ],
        compiler_params=pltpu.CompilerParams(use_tc_tiling_on_sc=False),
    )
    def gather_kernel(table_hbm, idx_hbm, out_hbm, idx_v, rows_v, sem):
        wid = lax.axis_index("s") * _NC + lax.axis_index("c")
        base = wid * _GRP_PER_W

        def chunk_body(i, carry):
            g0 = base + i * _KGRP
            pltpu.sync_copy(idx_hbm.at[pl.ds(seg_base + g0, _KGRP)], idx_v)
            copies = []
            for j in range(_KGRP):
                copies.append(
                    pltpu.async_copy(table_hbm.at[idx_v.at[j]], rows_v.at[j], sem)
                )
            for cp in copies:
                cp.wait()
            pltpu.sync_copy(rows_v, out_hbm.at[pl.ds(g0, _KGRP)])
            return carry

        lax.fori_loop(0, _NCHUNK, chunk_body, 0)

    return gather_kernel(table, idx_grp)


def _tanh_gate(t):
    return 0.5 * t + 0.5


T_STEP = 2


def _lstm_step(x4, h, c, wih, whh, b):
    x_t = jnp.concatenate([x4[:, :2 * E], x4[:, 2 * E:]], axis=0)
    gates = jnp.dot(x_t, wih, preferred_element_type=jnp.float32)
    gates = gates + jnp.dot(h, whh, preferred_element_type=jnp.float32)
    gates = gates + b
    i_g = _tanh_gate(jnp.tanh(gates[:, 0 * 2 * H:1 * 2 * H]))
    f_g = _tanh_gate(jnp.tanh(gates[:, 1 * 2 * H:2 * 2 * H]))
    g_g = jnp.tanh(gates[:, 2 * 2 * H:3 * 2 * H])
    o_g = _tanh_gate(jnp.tanh(gates[:, 3 * 2 * H:4 * 2 * H]))
    c_new = f_g * c + i_g * g_g
    h_new = o_g * jnp.tanh(c_new)
    return h_new, c_new


def _seg_body(emb_ref, hin_ref, cin_ref, wih_ref, whh_ref, b_ref,
              hout_ref, cout_ref, h_ref, c_ref):
    l = pl.program_id(0)

    @pl.when(l == 0)
    def _():
        h_ref[...] = hin_ref[...]
        c_ref[...] = cin_ref[...]

    h = h_ref[...]
    c = c_ref[...]
    wih = wih_ref[...]
    whh = whh_ref[...]
    b = b_ref[...]
    for k in range(T_STEP):
        h, c = _lstm_step(emb_ref[k], h, c, wih, whh, b)
    c_ref[...] = c
    h_ref[...] = h

    @pl.when(l == L_SEG // T_STEP - 1)
    def _():
        hout_ref[...] = h
        cout_ref[...] = c


def _lstm_seg(emb, h_in, c_in, wih2, whh2, bias2):
    return pl.pallas_call(
        _seg_body,
        grid=(L_SEG // T_STEP,),
        in_specs=[
            pl.BlockSpec((T_STEP, BP // 2, 4 * E), lambda l: (l, 0, 0)),
            pl.BlockSpec((BP, 2 * H), lambda l: (0, 0)),
            pl.BlockSpec((BP, 2 * H), lambda l: (0, 0)),
            pl.BlockSpec((2 * E, 8 * H), lambda l: (0, 0)),
            pl.BlockSpec((2 * H, 8 * H), lambda l: (0, 0)),
            pl.BlockSpec((1, 8 * H), lambda l: (0, 0)),
        ],
        out_specs=[
            pl.BlockSpec((BP, 2 * H), lambda l: (0, 0)),
            pl.BlockSpec((BP, 2 * H), lambda l: (0, 0)),
        ],
        out_shape=[
            jax.ShapeDtypeStruct((BP, 2 * H), jnp.float32),
            jax.ShapeDtypeStruct((BP, 2 * H), jnp.float32),
        ],
        scratch_shapes=[
            pltpu.VMEM((BP, 2 * H), jnp.float32),
            pltpu.VMEM((BP, 2 * H), jnp.float32),
        ],
    )(emb, h_in, c_in, wih2, whh2, bias2)


def _fc_body(h_ref, w1_ref, b1_ref, w2_ref, b2_ref, out_ref):
    z = jnp.dot(h_ref[...], w1_ref[...], preferred_element_type=jnp.float32)
    z = jnp.maximum(z + b1_ref[...], 0.0)
    out = jnp.dot(z, w2_ref[...], preferred_element_type=jnp.float32)
    out_ref[...] = out + b2_ref[...]


def _fc(h_fin, w1_2, b1_2, w2_2, b2_2):
    return pl.pallas_call(
        _fc_body,
        out_shape=jax.ShapeDtypeStruct((BP, 2 * C), jnp.float32),
    )(h_fin, w1_2, b1_2, w2_2, b2_2)


def _pack_block_diag(w, scale):
    K = w.shape[0]
    wg = (w * scale[None, :]).reshape(K, 4, H)
    z = jnp.zeros_like(wg)
    top = jnp.concatenate([wg, z], axis=2)
    bot = jnp.concatenate([z, wg], axis=2)
    return jnp.concatenate([top, bot], axis=0).reshape(2 * K, 4 * 2 * H)


def kernel(x, table, W_ih, W_hh, b_ih, b_hh, W1, b1, W2, b2):
    x = x.astype(jnp.int32)
    idx = x.T.reshape(L, 2, BP).transpose(0, 2, 1)
    idx_grp = idx.reshape(_NGRP, _GRP)

    gs = jnp.array([0.5, 0.5, 1.0, 0.5], jnp.float32)
    scale = jnp.repeat(gs, H)
    wih2 = _pack_block_diag(W_ih.T, scale)
    whh2 = _pack_block_diag(W_hh.T, scale)
    bias = ((b_ih + b_hh) * scale).reshape(4, H)
    bias2 = jnp.concatenate([bias, bias], axis=1).reshape(1, 8 * H)

    z32 = jnp.zeros((H, H2), jnp.float32)
    w1_2 = jnp.concatenate(
        [jnp.concatenate([W1.T, z32], axis=1),
         jnp.concatenate([z32, W1.T], axis=1)], axis=0)
    b1_2 = jnp.concatenate([b1, b1]).reshape(1, 2 * H2)
    zc = jnp.zeros((H2, C), jnp.float32)
    w2_2 = jnp.concatenate(
        [jnp.concatenate([W2.T, zc], axis=1),
         jnp.concatenate([zc, W2.T], axis=1)], axis=0)
    b2_2 = jnp.concatenate([b2, b2]).reshape(1, 2 * C)

    h = jnp.zeros((BP, 2 * H), jnp.float32)
    c = jnp.zeros((BP, 2 * H), jnp.float32)
    for seg in range(N_SEG):
        emb = _sc_gather_seg(table, idx_grp, seg)
        emb = emb.reshape(L_SEG, BP // 2, 4 * E)
        h, c = _lstm_seg(emb, h, c, wih2, whh2, bias2)

    out2 = _fc(h, w1_2, b1_2, w2_2, b2_2)
    out_nat = out2.reshape(2, BP // 2, 2 * C).transpose(1, 0, 2).reshape(BP, 2 * C)
    out = jnp.concatenate([out_nat[:, 0:C], out_nat[:, C:2 * C]], axis=0)
    return out[None]


# speedup vs baseline: 1.1447x; 1.0189x over previous
import functools

import jax
import jax.numpy as jnp
from jax import lax
from jax.experimental import pallas as pl
from jax.experimental.pallas import tpu as pltpu
from jax.experimental.pallas import tpu_sc as plsc

V = 1000000
E = 32
H = 64
C = 2
B = 4096
L = 200
H2 = H // 2
BP = B // 2

L_SEG = 40
N_SEG = L // L_SEG

_NC = 2
_NS = 16
_NW = _NC * _NS

_GRP = 128
_NGRP = (L * B) // _GRP
_NGRP_S = (L_SEG * B) // _GRP
_GRP_PER_W = _NGRP_S // _NW
_KGRP = 8
_NCHUNK = _GRP_PER_W // _KGRP


def _sc_gather_seg(table, idx_grp, seg):
    mesh = plsc.VectorSubcoreMesh(core_axis_name="c", subcore_axis_name="s")
    seg_base = seg * _NGRP_S

    @functools.partial(
        pl.kernel,
        mesh=mesh,
        out_type=jax.ShapeDtypeStruct((_NGRP_S, _GRP, E), jnp.float32),
        scratch_types=[
            pltpu.VMEM((_KGRP, _GRP), jnp.int32),
            pltpu.VMEM((_KGRP, _GRP, E), jnp.float32),
            pltpu.SemaphoreType.DMA,
        ],
        compiler_params=pltpu.CompilerParams(use_tc_tiling_on_sc=False),
    )
    def gather_kernel(table_hbm, idx_hbm, out_hbm, idx_v, rows_v, sem):
        wid = lax.axis_index("s") * _NC + lax.axis_index("c")
        base = wid * _GRP_PER_W

        def chunk_body(i, carry):
            g0 = base + i * _KGRP
            pltpu.sync_copy(idx_hbm.at[pl.ds(seg_base + g0, _KGRP)], idx_v)
            copies = []
            for j in range(_KGRP):
                copies.append(
                    pltpu.async_copy(table_hbm.at[idx_v.at[j]], rows_v.at[j], sem)
                )
            for cp in copies:
                cp.wait()
            pltpu.sync_copy(rows_v, out_hbm.at[pl.ds(g0, _KGRP)])
            return carry

        lax.fori_loop(0, _NCHUNK, chunk_body, 0)

    return gather_kernel(table, idx_grp)


def _tanh_gate(t):
    return 0.5 * t + 0.5


T_STEP = 4


def _lstm_step(x4, h, c, wih, whh, b):
    x_t = jnp.concatenate([x4[:, :2 * E], x4[:, 2 * E:]], axis=0)
    gates = jnp.dot(x_t, wih, preferred_element_type=jnp.float32)
    gates = gates + jnp.dot(h, whh, preferred_element_type=jnp.float32)
    gates = gates + b
    i_g = _tanh_gate(jnp.tanh(gates[:, 0 * 2 * H:1 * 2 * H]))
    f_g = _tanh_gate(jnp.tanh(gates[:, 1 * 2 * H:2 * 2 * H]))
    g_g = jnp.tanh(gates[:, 2 * 2 * H:3 * 2 * H])
    o_g = _tanh_gate(jnp.tanh(gates[:, 3 * 2 * H:4 * 2 * H]))
    c_new = f_g * c + i_g * g_g
    h_new = o_g * jnp.tanh(c_new)
    return h_new, c_new


def _seg_body(emb_ref, hin_ref, cin_ref, wih_ref, whh_ref, b_ref,
              hout_ref, cout_ref, h_ref, c_ref):
    l = pl.program_id(0)

    @pl.when(l == 0)
    def _():
        h_ref[...] = hin_ref[...]
        c_ref[...] = cin_ref[...]

    h = h_ref[...]
    c = c_ref[...]
    wih = wih_ref[...]
    whh = whh_ref[...]
    b = b_ref[...]
    for k in range(T_STEP):
        h, c = _lstm_step(emb_ref[k], h, c, wih, whh, b)
    c_ref[...] = c
    h_ref[...] = h

    @pl.when(l == L_SEG // T_STEP - 1)
    def _():
        hout_ref[...] = h
        cout_ref[...] = c


def _lstm_seg(emb, h_in, c_in, wih2, whh2, bias2):
    return pl.pallas_call(
        _seg_body,
        grid=(L_SEG // T_STEP,),
        in_specs=[
            pl.BlockSpec((T_STEP, BP // 2, 4 * E), lambda l: (l, 0, 0)),
            pl.BlockSpec((BP, 2 * H), lambda l: (0, 0)),
            pl.BlockSpec((BP, 2 * H), lambda l: (0, 0)),
            pl.BlockSpec((2 * E, 8 * H), lambda l: (0, 0)),
            pl.BlockSpec((2 * H, 8 * H), lambda l: (0, 0)),
            pl.BlockSpec((1, 8 * H), lambda l: (0, 0)),
        ],
        out_specs=[
            pl.BlockSpec((BP, 2 * H), lambda l: (0, 0)),
            pl.BlockSpec((BP, 2 * H), lambda l: (0, 0)),
        ],
        out_shape=[
            jax.ShapeDtypeStruct((BP, 2 * H), jnp.float32),
            jax.ShapeDtypeStruct((BP, 2 * H), jnp.float32),
        ],
        scratch_shapes=[
            pltpu.VMEM((BP, 2 * H), jnp.float32),
            pltpu.VMEM((BP, 2 * H), jnp.float32),
        ],
    )(emb, h_in, c_in, wih2, whh2, bias2)


def _fc_body(h_ref, w1_ref, b1_ref, w2_ref, b2_ref, out_ref):
    z = jnp.dot(h_ref[...], w1_ref[...], preferred_element_type=jnp.float32)
    z = jnp.maximum(z + b1_ref[...], 0.0)
    out = jnp.dot(z, w2_ref[...], preferred_element_type=jnp.float32)
    out_ref[...] = out + b2_ref[...]


def _fc(h_fin, w1_2, b1_2, w2_2, b2_2):
    return pl.pallas_call(
        _fc_body,
        out_shape=jax.ShapeDtypeStruct((BP, 2 * C), jnp.float32),
    )(h_fin, w1_2, b1_2, w2_2, b2_2)


def _pack_block_diag(w, scale):
    K = w.shape[0]
    wg = (w * scale[None, :]).reshape(K, 4, H)
    z = jnp.zeros_like(wg)
    top = jnp.concatenate([wg, z], axis=2)
    bot = jnp.concatenate([z, wg], axis=2)
    return jnp.concatenate([top, bot], axis=0).reshape(2 * K, 4 * 2 * H)


def kernel(x, table, W_ih, W_hh, b_ih, b_hh, W1, b1, W2, b2):
    x = x.astype(jnp.int32)
    idx = x.T.reshape(L, 2, BP).transpose(0, 2, 1)
    idx_grp = idx.reshape(_NGRP, _GRP)

    gs = jnp.array([0.5, 0.5, 1.0, 0.5], jnp.float32)
    scale = jnp.repeat(gs, H)
    wih2 = _pack_block_diag(W_ih.T, scale)
    whh2 = _pack_block_diag(W_hh.T, scale)
    bias = ((b_ih + b_hh) * scale).reshape(4, H)
    bias2 = jnp.concatenate([bias, bias], axis=1).reshape(1, 8 * H)

    z32 = jnp.zeros((H, H2), jnp.float32)
    w1_2 = jnp.concatenate(
        [jnp.concatenate([W1.T, z32], axis=1),
         jnp.concatenate([z32, W1.T], axis=1)], axis=0)
    b1_2 = jnp.concatenate([b1, b1]).reshape(1, 2 * H2)
    zc = jnp.zeros((H2, C), jnp.float32)
    w2_2 = jnp.concatenate(
        [jnp.concatenate([W2.T, zc], axis=1),
         jnp.concatenate([zc, W2.T], axis=1)], axis=0)
    b2_2 = jnp.concatenate([b2, b2]).reshape(1, 2 * C)

    h = jnp.zeros((BP, 2 * H), jnp.float32)
    c = jnp.zeros((BP, 2 * H), jnp.float32)
    for seg in range(N_SEG):
        emb = _sc_gather_seg(table, idx_grp, seg)
        emb = emb.reshape(L_SEG, BP // 2, 4 * E)
        h, c = _lstm_seg(emb, h, c, wih2, whh2, bias2)

    out2 = _fc(h, w1_2, b1_2, w2_2, b2_2)
    out_nat = out2.reshape(2, BP // 2, 2 * C).transpose(1, 0, 2).reshape(BP, 2 * C)
    out = jnp.concatenate([out_nat[:, 0:C], out_nat[:, C:2 * C]], axis=0)
    return out[None]
